# baseline (speedup 1.0000x reference)
import functools

import jax
import jax.numpy as jnp
from jax import lax
from jax.experimental import pallas as pl
from jax.experimental.pallas import tpu as pltpu
from jax.experimental.pallas import tpu_sc as plsc

L = 16


def _make_lookup(R, C, V, num_cores, num_subcores):
    NW = num_cores * num_subcores
    cols_w = C // NW
    vregs_row = cols_w // L
    mesh = plsc.VectorSubcoreMesh(core_axis_name="c", subcore_axis_name="s")

    @functools.partial(
        pl.kernel,
        mesh=mesh,
        out_type=jax.ShapeDtypeStruct((R, C), jnp.int32),
        scratch_types=[
            pltpu.VMEM((V,), jnp.int32),
            pltpu.VMEM((R, cols_w), jnp.int32),
            pltpu.VMEM((R, cols_w), jnp.int32),
            pltpu.SemaphoreType.DMA,
            pltpu.SemaphoreType.DMA,
            pltpu.SemaphoreType.DMA,
        ],
        compiler_params=pltpu.CompilerParams(needs_layout_passes=False),
    )
    def k(tok_hbm, tab_hbm, out_hbm, tab_v, tok_v, out_v, sem_tab, sem_tok, sem_out):
        wid = lax.axis_index("s") * num_cores + lax.axis_index("c")
        c0 = wid * cols_w

        nchunk = 5
        rchunk = R // nchunk
        tab_cp = pltpu.async_copy(tab_hbm, tab_v, sem_tab)
        tok_cps = [
            pltpu.async_copy(
                tok_hbm.at[pl.ds(ci * rchunk, rchunk), pl.ds(c0, cols_w)],
                tok_v.at[pl.ds(ci * rchunk, rchunk)],
                sem_tok,
            )
            for ci in range(nchunk)
        ]
        tab_cp.wait()

        out_cps = []
        for ci in range(nchunk):
            r0 = ci * rchunk
            tok_cps[ci].wait()

            @plsc.parallel_loop(r0, r0 + rchunk, 1, unroll=4)
            def body(r):
                for j in range(vregs_row):
                    t = tok_v[r, pl.ds(j * L, L)]
                    idx = lax.shift_right_logical(t, 1)
                    out_v[r, pl.ds(j * L, L)] = plsc.load_gather(tab_v, [idx])

            out_cps.append(
                pltpu.async_copy(
                    out_v.at[pl.ds(r0, rchunk)],
                    out_hbm.at[pl.ds(r0, rchunk), pl.ds(c0, cols_w)],
                    sem_out,
                )
            )
        for cp in out_cps:
            cp.wait()

    return k


def kernel(tokens, table):
    R, C = tokens.shape
    V = table.shape[0]
    info = plsc.get_sparse_core_info()
    k = _make_lookup(C, R, V, info.num_cores, info.num_subcores)
    out_t = k(tokens.T, table)
    return out_t.T


# speedup vs baseline: 1.0753x; 1.0753x over previous
import functools

import jax
import jax.numpy as jnp
from jax import lax
from jax.experimental import pallas as pl
from jax.experimental.pallas import tpu as pltpu
from jax.experimental.pallas import tpu_sc as plsc

L = 16


def _make_lookup(R, C, V, num_cores, num_subcores):
    NW = num_cores * num_subcores
    cols_w = C // NW
    vregs_row = cols_w // L
    mesh = plsc.VectorSubcoreMesh(core_axis_name="c", subcore_axis_name="s")

    @functools.partial(
        pl.kernel,
        mesh=mesh,
        out_type=jax.ShapeDtypeStruct((R, C), jnp.int32),
        scratch_types=[
            pltpu.VMEM((V,), jnp.int32),
            pltpu.VMEM((R, cols_w), jnp.int32),
            pltpu.VMEM((R, cols_w), jnp.int32),
            pltpu.SemaphoreType.DMA,
            pltpu.SemaphoreType.DMA,
            pltpu.SemaphoreType.DMA,
        ],
        compiler_params=pltpu.CompilerParams(needs_layout_passes=False),
    )
    def k(tok_hbm, tab_hbm, out_hbm, tab_v, tok_v, out_v, sem_tab, sem_tok, sem_out):
        wid = lax.axis_index("s") * num_cores + lax.axis_index("c")
        c0 = wid * cols_w

        tab_cp = pltpu.async_copy(tab_hbm, tab_v, sem_tab)
        tok_cp = pltpu.async_copy(tok_hbm.at[:, pl.ds(c0, cols_w)], tok_v, sem_tok)
        tab_cp.wait()
        tok_cp.wait()

        @plsc.parallel_loop(0, R, 1, unroll=2)
        def body(r):
            for j in range(vregs_row):
                t = tok_v[r, pl.ds(j * L, L)]
                idx = lax.shift_right_logical(t, 1)
                out_v[r, pl.ds(j * L, L)] = plsc.load_gather(tab_v, [idx])

        pltpu.async_copy(
            out_v, out_hbm.at[:, pl.ds(c0, cols_w)], sem_out
        ).wait()

    return k


def kernel(tokens, table):
    R, C = tokens.shape
    V = table.shape[0]
    info = plsc.get_sparse_core_info()
    k = _make_lookup(C, R, V, info.num_cores, info.num_subcores)
    out_t = k(tokens.T, table)
    return out_t.T


# speedup vs baseline: 1.0804x; 1.0048x over previous
import functools

import jax
import jax.numpy as jnp
from jax import lax
from jax.experimental import pallas as pl
from jax.experimental.pallas import tpu as pltpu
from jax.experimental.pallas import tpu_sc as plsc

L = 16


def _make_lookup(R, C, V, num_cores, num_subcores):
    NW = num_cores * num_subcores
    cols_w = C // NW
    vregs_row = cols_w // L
    mesh = plsc.VectorSubcoreMesh(core_axis_name="c", subcore_axis_name="s")

    @functools.partial(
        pl.kernel,
        mesh=mesh,
        out_type=jax.ShapeDtypeStruct((R, C), jnp.int32),
        scratch_types=[
            pltpu.VMEM((V,), jnp.int32),
            pltpu.VMEM((R, cols_w), jnp.int32),
            pltpu.VMEM((R, cols_w), jnp.int32),
            pltpu.SemaphoreType.DMA,
            pltpu.SemaphoreType.DMA,
            pltpu.SemaphoreType.DMA,
        ],
        compiler_params=pltpu.CompilerParams(needs_layout_passes=False),
    )
    def k(tok_hbm, tab_hbm, out_hbm, tab_v, tok_v, out_v, sem_tab, sem_tok, sem_out):
        wid = lax.axis_index("s") * num_cores + lax.axis_index("c")
        c0 = wid * cols_w

        tab_cp = pltpu.async_copy(tab_hbm, tab_v, sem_tab)
        tok_cp = pltpu.async_copy(tok_hbm.at[:, pl.ds(c0, cols_w)], tok_v, sem_tok)
        tab_cp.wait()
        tok_cp.wait()

        @plsc.parallel_loop(0, R, 1, unroll=4)
        def body(r):
            for j in range(vregs_row):
                t = tok_v[r, pl.ds(j * L, L)]
                idx = lax.shift_right_logical(t, 1)
                out_v[r, pl.ds(j * L, L)] = plsc.load_gather(tab_v, [idx])

        pltpu.async_copy(
            out_v, out_hbm.at[:, pl.ds(c0, cols_w)], sem_out
        ).wait()

    return k


def kernel(tokens, table):
    R, C = tokens.shape
    V = table.shape[0]
    info = plsc.get_sparse_core_info()
    k = _make_lookup(C, R, V, info.num_cores, info.num_subcores)
    out_t = k(tokens.T, table)
    return out_t.T


# speedup vs baseline: 1.0920x; 1.0107x over previous
import functools

import jax
import jax.numpy as jnp
from jax import lax
from jax.experimental import pallas as pl
from jax.experimental.pallas import tpu as pltpu
from jax.experimental.pallas import tpu_sc as plsc

L = 16


def _make_lookup(R, C, V, num_cores, num_subcores):
    NW = num_cores * num_subcores
    cols_w = C // NW
    vregs_row = cols_w // L
    mesh = plsc.VectorSubcoreMesh(core_axis_name="c", subcore_axis_name="s")

    @functools.partial(
        pl.kernel,
        mesh=mesh,
        out_type=jax.ShapeDtypeStruct((R, C), jnp.int32),
        scratch_types=[
            pltpu.VMEM((V,), jnp.int32),
            pltpu.VMEM((R, cols_w), jnp.int32),
            pltpu.VMEM((R, cols_w), jnp.int32),
            pltpu.SemaphoreType.DMA,
            pltpu.SemaphoreType.DMA,
            pltpu.SemaphoreType.DMA,
        ],
        compiler_params=pltpu.CompilerParams(needs_layout_passes=False),
    )
    def k(tok_hbm, tab_hbm, out_hbm, tab_v, tok_v, out_v, sem_tab, sem_tok, sem_out):
        wid = lax.axis_index("s") * num_cores + lax.axis_index("c")
        c0 = wid * cols_w

        tab_cp = pltpu.async_copy(tab_hbm, tab_v, sem_tab)
        tok_cp = pltpu.async_copy(tok_hbm.at[:, pl.ds(c0, cols_w)], tok_v, sem_tok)
        tab_cp.wait()
        tok_cp.wait()

        nchunk = 5
        rchunk = R // nchunk

        def chunk_body(ci, carry):
            r0 = ci * rchunk

            @plsc.parallel_loop(r0, r0 + rchunk, 1, unroll=4)
            def body(r):
                for j in range(vregs_row):
                    t = tok_v[r, pl.ds(j * L, L)]
                    idx = lax.shift_right_logical(t, 1)
                    out_v[r, pl.ds(j * L, L)] = plsc.load_gather(tab_v, [idx])

            pltpu.async_copy(
                out_v.at[pl.ds(r0, rchunk)],
                out_hbm.at[pl.ds(r0, rchunk), pl.ds(c0, cols_w)],
                sem_out,
            )
            return carry

        lax.fori_loop(0, nchunk, chunk_body, 0)
        for ci in range(nchunk):
            pltpu.make_async_copy(
                out_v.at[pl.ds(ci * rchunk, rchunk)],
                out_hbm.at[pl.ds(ci * rchunk, rchunk), pl.ds(c0, cols_w)],
                sem_out,
            ).wait()

    return k


def kernel(tokens, table):
    R, C = tokens.shape
    V = table.shape[0]
    info = plsc.get_sparse_core_info()
    k = _make_lookup(C, R, V, info.num_cores, info.num_subcores)
    out_t = k(tokens.T, table)
    return out_t.T
